# Initial kernel scaffold; baseline (speedup 1.0000x reference)
import jax
import jax.numpy as jnp
from jax.experimental import pallas as pl


def kernel(input, edge, edge_embed, a, a_2):
    raise NotImplementedError("write your pallas kernel here")


# speedup vs baseline: 3.2581x; 3.2581x over previous
import functools

import jax
import jax.numpy as jnp
from jax import lax
from jax.experimental import pallas as pl
from jax.experimental.pallas import tpu as pltpu
from jax.experimental.pallas import tpu_sc as plsc

N = 10000
E = 320000
DIN = 128
DOUT = 128
ALPHA = 0.2

CH = 128
NSUB = CH // 128
NCHUNKS = E // CH
NW = 32
NTILE = 16
ROWS_PER_TILE = 624
ZROWS = 16
NRS = 1280
RS_ROWS_PER_TILE = NRS // NTILE

BASE_CHUNKS = NCHUNKS // NW
EXTRA = NCHUNKS % NW


def _proj_body(x_ref, w_ref, a2_ref, h_ref, s_ref):
    h = jnp.dot(x_ref[...], w_ref[...], preferred_element_type=jnp.float32)
    h_ref[...] = h
    s_ref[...] = jnp.sum(h * a2_ref[...], axis=1, keepdims=True)


def _projection(x, wt, a2, block_rows):
    rows = x.shape[0]
    grid = rows // block_rows
    return pl.pallas_call(
        _proj_body,
        grid=(grid,),
        in_specs=[
            pl.BlockSpec((block_rows, DIN), lambda i: (i, 0)),
            pl.BlockSpec((DIN, DOUT), lambda i: (0, 0)),
            pl.BlockSpec((1, DOUT), lambda i: (0, 0)),
        ],
        out_specs=[
            pl.BlockSpec((block_rows, DOUT), lambda i: (i, 0)),
            pl.BlockSpec((block_rows, 1), lambda i: (i, 0)),
        ],
        out_shape=[
            jax.ShapeDtypeStruct((rows, DOUT), jnp.float32),
            jax.ShapeDtypeStruct((rows, 1), jnp.float32),
        ],
    )(x, wt, a2)


def _sc_body(src_hbm, semb_hbm, sproj_hbm, emb_hbm, num_out, rs_out,
             sproj_v, src_v, semb_v, emb_v, evalue_v, idx16_v,
             num_sh, rs8_sh):
    cid = lax.axis_index("c")
    sid = lax.axis_index("s")
    wid = sid * 2 + cid

    zero16 = jnp.zeros((16,), jnp.float32)

    def zfill(i, carry):
        for j in range(8):
            emb_v[i, pl.ds(j * 16, 16)] = zero16
            evalue_v[i, pl.ds(j * 16, 16)] = zero16
        return carry

    lax.fori_loop(0, ZROWS, zfill, 0)
    row0 = pl.multiple_of(sid * ROWS_PER_TILE, 8)
    nz = ROWS_PER_TILE // ZROWS + jnp.where(sid == NTILE - 1, 1, 0)

    def zcopy(t, carry):
        r = pl.multiple_of(row0 + t * ZROWS, 8)
        pltpu.sync_copy(emb_v.at[pl.ds(0, ZROWS)], num_sh.at[pl.ds(r, ZROWS)])
        return carry

    lax.fori_loop(0, nz, zcopy, 0)
    rrow0 = pl.multiple_of(sid * RS_ROWS_PER_TILE, 8)

    def zcopy_rs(t, carry):
        r = pl.multiple_of(rrow0 + t * ZROWS, 8)
        pltpu.sync_copy(emb_v.at[pl.ds(0, ZROWS)], rs8_sh.at[pl.ds(r, ZROWS)])
        return carry

    lax.fori_loop(0, RS_ROWS_PER_TILE // ZROWS, zcopy_rs, 0)

    pltpu.sync_copy(sproj_hbm, sproj_v)
    plsc.subcore_barrier()

    nch = BASE_CHUNKS + jnp.where(wid < EXTRA, 1, 0)

    def chunk_body(t, carry):
        c = wid + t * NW
        base_e = pl.multiple_of(c * CH, 8)
        pltpu.sync_copy(src_hbm.at[c], src_v)
        pltpu.sync_copy(semb_hbm.at[pl.ds(base_e, CH)], semb_v)
        pltpu.sync_copy(emb_hbm.at[pl.ds(base_e, CH)], emb_v)

        def gbody(g, inner):
            col = g * 16
            idx = src_v[0, pl.ds(col, 16)]
            sp = plsc.load_gather(sproj_v, [idx])
            s = sp + semb_v[pl.ds(col, 16)]
            e = jnp.exp(jnp.minimum(-s, (-ALPHA) * s))
            qoff = (idx & 7) * 16
            idx16_v[...] = lax.shift_right_logical(idx, 3)
            for k in range(16):
                i = col + k
                ei = e[k]
                for j in range(8):
                    emb_v[i, pl.ds(j * 16, 16)] = (
                        emb_v[i, pl.ds(j * 16, 16)] * ei)
                evalue_v[k, pl.ds(qoff[k], 16)] = jnp.broadcast_to(ei, (16,))
            pltpu.sync_copy(evalue_v, rs8_sh.at[idx16_v], add=True)
            for k in range(16):
                evalue_v[k, pl.ds(qoff[k], 16)] = zero16
            return inner

        lax.fori_loop(0, CH // 16, gbody, 0)

        pltpu.sync_copy(emb_v, num_sh.at[src_v.at[0]], add=True)
        return carry

    lax.fori_loop(0, nch, chunk_body, 0)

    plsc.subcore_barrier()
    pltpu.sync_copy(num_sh.at[pl.ds(row0, ROWS_PER_TILE)],
                    num_out.at[cid, pl.ds(row0, ROWS_PER_TILE)])
    pltpu.sync_copy(rs8_sh.at[pl.ds(rrow0, RS_ROWS_PER_TILE)],
                    rs_out.at[cid, pl.ds(rrow0, RS_ROWS_PER_TILE)])

    @pl.when(sid == NTILE - 1)
    def _tail():
        r = NTILE * ROWS_PER_TILE
        pltpu.sync_copy(num_sh.at[pl.ds(r, N - r)],
                        num_out.at[cid, pl.ds(r, N - r)])


def _sc_segment(src2d, semb, sproj, emb_proj):
    mesh = plsc.VectorSubcoreMesh(core_axis_name="c", subcore_axis_name="s")
    f = pl.kernel(
        _sc_body,
        out_type=[
            jax.ShapeDtypeStruct((2, N, DOUT), jnp.float32),
            jax.ShapeDtypeStruct((2, NRS, 128), jnp.float32),
        ],
        mesh=mesh,
        compiler_params=pltpu.CompilerParams(needs_layout_passes=False),
        scratch_types=[
            pltpu.VMEM((N,), jnp.float32),
            pltpu.VMEM((NSUB, 128), jnp.int32),
            pltpu.VMEM((CH,), jnp.float32),
            pltpu.VMEM((CH, DOUT), jnp.float32),
            pltpu.VMEM((16, 128), jnp.float32),
            pltpu.VMEM((16,), jnp.int32),
            pltpu.VMEM_SHARED((N, DOUT), jnp.float32),
            pltpu.VMEM_SHARED((NRS, 128), jnp.float32),
        ],
    )
    return f(src2d, semb, sproj, emb_proj)


def _comb_body(h_ref, n0_ref, n1_ref, r0_ref, r1_ref, o_ref):
    rs = r0_ref[:, 0:1] + r1_ref[:, 0:1]
    num = n0_ref[...] + n1_ref[...]
    o_ref[...] = jnp.where(rs > 0.0, h_ref[...] + num / rs, 0.0)


def _combine(h_proj, num, rs, block_rows):
    grid = N // block_rows
    rspec = pl.BlockSpec((block_rows, DOUT), lambda i: (i, 0))
    sspec = pl.BlockSpec((block_rows, 16), lambda i: (i, 0))
    return pl.pallas_call(
        _comb_body,
        grid=(grid,),
        in_specs=[rspec, rspec, rspec, sspec, sspec],
        out_specs=rspec,
        out_shape=jax.ShapeDtypeStruct((N, DOUT), jnp.float32),
    )(h_proj, num[0], num[1], rs[0], rs[1])


@jax.jit
def kernel(input, edge, edge_embed, a, a_2):
    w1t = a[:, :DIN].T
    w2t = a[:, DIN:].T
    src3d = edge[0].reshape(NCHUNKS, NSUB, 128)

    h_proj, sproj = _projection(input, w1t, a_2, 2000)
    emb_proj, semb = _projection(edge_embed, w2t, a_2, 2560)

    num, rs8 = _sc_segment(src3d, semb.reshape(E), sproj.reshape(N), emb_proj)
    rs = rs8.reshape(2, NRS * 8, 16)[:, :N, :]
    return _combine(h_proj, num, rs, 2000)


# speedup vs baseline: 3.7130x; 1.1396x over previous
import functools

import jax
import jax.numpy as jnp
from jax import lax
from jax.experimental import pallas as pl
from jax.experimental.pallas import tpu as pltpu
from jax.experimental.pallas import tpu_sc as plsc

N = 10000
E = 320000
DIN = 128
DOUT = 128
ALPHA = 0.2

CH = 128
NSUB = CH // 128
NCHUNKS = E // CH
NW = 32
NTILE = 16
ROWS_PER_TILE = 624
ZROWS = 16
NRS = 1280
RS_ROWS_PER_TILE = NRS // NTILE

BASE_CHUNKS = NCHUNKS // NW
EXTRA = NCHUNKS % NW


def _proj_body(x_ref, w_ref, a2_ref, h_ref, s_ref):
    h = jnp.dot(x_ref[...], w_ref[...], preferred_element_type=jnp.float32)
    h_ref[...] = h
    s_ref[...] = jnp.sum(h * a2_ref[...], axis=1, keepdims=True)


def _projection(x, wt, a2, block_rows):
    rows = x.shape[0]
    grid = rows // block_rows
    return pl.pallas_call(
        _proj_body,
        grid=(grid,),
        in_specs=[
            pl.BlockSpec((block_rows, DIN), lambda i: (i, 0)),
            pl.BlockSpec((DIN, DOUT), lambda i: (0, 0)),
            pl.BlockSpec((1, DOUT), lambda i: (0, 0)),
        ],
        out_specs=[
            pl.BlockSpec((block_rows, DOUT), lambda i: (i, 0)),
            pl.BlockSpec((block_rows, 1), lambda i: (i, 0)),
        ],
        out_shape=[
            jax.ShapeDtypeStruct((rows, DOUT), jnp.float32),
            jax.ShapeDtypeStruct((rows, 1), jnp.float32),
        ],
    )(x, wt, a2)


def _sc_body(src_hbm, semb_hbm, sproj_hbm, emb_hbm, num_out, rs_out,
             sproj_v, src_v, semb_v, emb_v, evalue_v, idx64_v, qbuf_v,
             sem1, sem2, sem3, num_sh, rs8_sh):
    cid = lax.axis_index("c")
    sid = lax.axis_index("s")
    wid = sid * 2 + cid

    zero16 = jnp.zeros((16,), jnp.float32)

    def zfill(i, carry):
        for j in range(8):
            emb_v[i, pl.ds(j * 16, 16)] = zero16
        return carry

    lax.fori_loop(0, ZROWS, zfill, 0)

    def zfill2(i, carry):
        for j in range(8):
            evalue_v[i, pl.ds(j * 16, 16)] = zero16
        return carry

    lax.fori_loop(0, 64, zfill2, 0)
    row0 = pl.multiple_of(sid * ROWS_PER_TILE, 8)
    nz = ROWS_PER_TILE // ZROWS + jnp.where(sid == NTILE - 1, 1, 0)

    def zcopy(t, carry):
        r = pl.multiple_of(row0 + t * ZROWS, 8)
        pltpu.sync_copy(emb_v.at[pl.ds(0, ZROWS)], num_sh.at[pl.ds(r, ZROWS)])
        return carry

    lax.fori_loop(0, nz, zcopy, 0)
    rrow0 = pl.multiple_of(sid * RS_ROWS_PER_TILE, 8)

    def zcopy_rs(t, carry):
        r = pl.multiple_of(rrow0 + t * ZROWS, 8)
        pltpu.sync_copy(emb_v.at[pl.ds(0, ZROWS)], rs8_sh.at[pl.ds(r, ZROWS)])
        return carry

    lax.fori_loop(0, RS_ROWS_PER_TILE // ZROWS, zcopy_rs, 0)

    pltpu.sync_copy(sproj_hbm, sproj_v)
    plsc.subcore_barrier()

    nch = BASE_CHUNKS + jnp.where(wid < EXTRA, 1, 0)

    def chunk_body(t, carry):
        c = wid + t * NW
        base_e = pl.multiple_of(c * CH, 8)
        cp1 = pltpu.async_copy(src_hbm.at[c], src_v, sem1)
        cp2 = pltpu.async_copy(semb_hbm.at[pl.ds(base_e, CH)], semb_v, sem2)
        cp3 = pltpu.async_copy(emb_hbm.at[pl.ds(base_e, CH)], emb_v, sem3)
        cp1.wait()
        cp2.wait()
        cp3.wait()

        def bbody(b, inner):
            col0 = b * 64
            for g in range(4):
                col = col0 + g * 16
                idx = src_v[0, pl.ds(col, 16)]
                sp = plsc.load_gather(sproj_v, [idx])
                s = sp + semb_v[pl.ds(col, 16)]
                e = jnp.exp(jnp.minimum(-s, (-ALPHA) * s))
                qoff = (idx & 7) * 16
                idx64_v[pl.ds(g * 16, 16)] = lax.shift_right_logical(idx, 3)
                qbuf_v[pl.ds(g * 16, 16)] = qoff
                for k in range(16):
                    i = col + k
                    ei = e[k]
                    for j in range(8):
                        emb_v[i, pl.ds(j * 16, 16)] = (
                            emb_v[i, pl.ds(j * 16, 16)] * ei)
                    evalue_v[g * 16 + k, pl.ds(qoff[k], 16)] = (
                        jnp.broadcast_to(ei, (16,)))
            pltpu.sync_copy(evalue_v, rs8_sh.at[idx64_v], add=True)
            for g in range(4):
                qv = qbuf_v[pl.ds(g * 16, 16)]
                for k in range(16):
                    evalue_v[g * 16 + k, pl.ds(qv[k], 16)] = zero16
            return inner

        lax.fori_loop(0, CH // 64, bbody, 0)

        pltpu.sync_copy(emb_v, num_sh.at[src_v.at[0]], add=True)
        return carry

    lax.fori_loop(0, nch, chunk_body, 0)

    plsc.subcore_barrier()
    pltpu.sync_copy(num_sh.at[pl.ds(row0, ROWS_PER_TILE)],
                    num_out.at[cid, pl.ds(row0, ROWS_PER_TILE)])
    pltpu.sync_copy(rs8_sh.at[pl.ds(rrow0, RS_ROWS_PER_TILE)],
                    rs_out.at[cid, pl.ds(rrow0, RS_ROWS_PER_TILE)])

    @pl.when(sid == NTILE - 1)
    def _tail():
        r = NTILE * ROWS_PER_TILE
        pltpu.sync_copy(num_sh.at[pl.ds(r, N - r)],
                        num_out.at[cid, pl.ds(r, N - r)])


def _sc_segment(src2d, semb, sproj, emb_proj):
    mesh = plsc.VectorSubcoreMesh(core_axis_name="c", subcore_axis_name="s")
    f = pl.kernel(
        _sc_body,
        out_type=[
            jax.ShapeDtypeStruct((2, N, DOUT), jnp.float32),
            jax.ShapeDtypeStruct((2, NRS, 128), jnp.float32),
        ],
        mesh=mesh,
        compiler_params=pltpu.CompilerParams(needs_layout_passes=False),
        scratch_types=[
            pltpu.VMEM((N,), jnp.float32),
            pltpu.VMEM((NSUB, 128), jnp.int32),
            pltpu.VMEM((CH,), jnp.float32),
            pltpu.VMEM((CH, DOUT), jnp.float32),
            pltpu.VMEM((64, 128), jnp.float32),
            pltpu.VMEM((64,), jnp.int32),
            pltpu.VMEM((64,), jnp.int32),
            pltpu.SemaphoreType.DMA,
            pltpu.SemaphoreType.DMA,
            pltpu.SemaphoreType.DMA,
            pltpu.VMEM_SHARED((N, DOUT), jnp.float32),
            pltpu.VMEM_SHARED((NRS, 128), jnp.float32),
        ],
    )
    return f(src2d, semb, sproj, emb_proj)


def _comb_body(h_ref, n0_ref, n1_ref, r0_ref, r1_ref, o_ref):
    rs = r0_ref[:, 0:1] + r1_ref[:, 0:1]
    num = n0_ref[...] + n1_ref[...]
    o_ref[...] = jnp.where(rs > 0.0, h_ref[...] + num / rs, 0.0)


def _combine(h_proj, num, rs, block_rows):
    grid = N // block_rows
    rspec = pl.BlockSpec((block_rows, DOUT), lambda i: (i, 0))
    sspec = pl.BlockSpec((block_rows, 16), lambda i: (i, 0))
    return pl.pallas_call(
        _comb_body,
        grid=(grid,),
        in_specs=[rspec, rspec, rspec, sspec, sspec],
        out_specs=rspec,
        out_shape=jax.ShapeDtypeStruct((N, DOUT), jnp.float32),
    )(h_proj, num[0], num[1], rs[0], rs[1])


@jax.jit
def kernel(input, edge, edge_embed, a, a_2):
    w1t = a[:, :DIN].T
    w2t = a[:, DIN:].T
    src3d = edge[0].reshape(NCHUNKS, NSUB, 128)

    h_proj, sproj = _projection(input, w1t, a_2, 2000)
    emb_proj, semb = _projection(edge_embed, w2t, a_2, 2560)

    num, rs8 = _sc_segment(src3d, semb.reshape(E), sproj.reshape(N), emb_proj)
    rs = rs8.reshape(2, NRS * 8, 16)[:, :N, :]
    return _combine(h_proj, num, rs, 2000)


# speedup vs baseline: 3.7173x; 1.0012x over previous
import functools

import jax
import jax.numpy as jnp
from jax import lax
from jax.experimental import pallas as pl
from jax.experimental.pallas import tpu as pltpu
from jax.experimental.pallas import tpu_sc as plsc

N = 10000
E = 320000
DIN = 128
DOUT = 128
ALPHA = 0.2

CH = 128
NSUB = CH // 128
NCHUNKS = E // CH
NW = 32
NTILE = 16
ROWS_PER_TILE = 624
ZROWS = 16
NRS = 1280
RS_ROWS_PER_TILE = NRS // NTILE

BASE_CHUNKS = NCHUNKS // NW
EXTRA = NCHUNKS % NW


def _proj_body(x_ref, w_ref, a2_ref, h_ref, s_ref):
    h = jnp.dot(x_ref[...], w_ref[...], preferred_element_type=jnp.float32)
    h_ref[...] = h
    s_ref[...] = jnp.sum(h * a2_ref[...], axis=1, keepdims=True)


def _projection(x, wt, a2, block_rows):
    rows = x.shape[0]
    grid = rows // block_rows
    return pl.pallas_call(
        _proj_body,
        grid=(grid,),
        in_specs=[
            pl.BlockSpec((block_rows, DIN), lambda i: (i, 0)),
            pl.BlockSpec((DIN, DOUT), lambda i: (0, 0)),
            pl.BlockSpec((1, DOUT), lambda i: (0, 0)),
        ],
        out_specs=[
            pl.BlockSpec((block_rows, DOUT), lambda i: (i, 0)),
            pl.BlockSpec((block_rows, 1), lambda i: (i, 0)),
        ],
        out_shape=[
            jax.ShapeDtypeStruct((rows, DOUT), jnp.float32),
            jax.ShapeDtypeStruct((rows, 1), jnp.float32),
        ],
    )(x, wt, a2)


def _sc_body(src_hbm, semb_hbm, sproj_hbm, emb_hbm, num_out, rs_out,
             sproj_v, src_v, semb_v, emb_v, evalue_v, idx64_v, qbuf_v,
             sem1, sem2, sem3, semn, num_sh, rs8_sh):
    cid = lax.axis_index("c")
    sid = lax.axis_index("s")
    wid = sid * 2 + cid

    zero16 = jnp.zeros((16,), jnp.float32)

    def zfill(i, carry):
        for j in range(8):
            emb_v[i, pl.ds(j * 16, 16)] = zero16
        return carry

    lax.fori_loop(0, ZROWS, zfill, 0)

    def zfill2(i, carry):
        for j in range(8):
            evalue_v[i, pl.ds(j * 16, 16)] = zero16
        return carry

    lax.fori_loop(0, 64, zfill2, 0)
    row0 = pl.multiple_of(sid * ROWS_PER_TILE, 8)
    nz = ROWS_PER_TILE // ZROWS + jnp.where(sid == NTILE - 1, 1, 0)

    def zcopy(t, carry):
        r = pl.multiple_of(row0 + t * ZROWS, 8)
        pltpu.sync_copy(emb_v.at[pl.ds(0, ZROWS)], num_sh.at[pl.ds(r, ZROWS)])
        return carry

    lax.fori_loop(0, nz, zcopy, 0)
    rrow0 = pl.multiple_of(sid * RS_ROWS_PER_TILE, 8)

    def zcopy_rs(t, carry):
        r = pl.multiple_of(rrow0 + t * ZROWS, 8)
        pltpu.sync_copy(emb_v.at[pl.ds(0, ZROWS)], rs8_sh.at[pl.ds(r, ZROWS)])
        return carry

    lax.fori_loop(0, RS_ROWS_PER_TILE // ZROWS, zcopy_rs, 0)

    pltpu.sync_copy(sproj_hbm, sproj_v)
    plsc.subcore_barrier()

    nch = BASE_CHUNKS + jnp.where(wid < EXTRA, 1, 0)

    def chunk_body(t, carry):
        c = wid + t * NW
        base_e = pl.multiple_of(c * CH, 8)

        @pl.when(t > 0)
        def _drain_prev():
            pltpu.make_async_copy(emb_v, num_sh.at[src_v.at[0]], semn).wait()

        cp1 = pltpu.async_copy(src_hbm.at[c], src_v, sem1)
        cp2 = pltpu.async_copy(semb_hbm.at[pl.ds(base_e, CH)], semb_v, sem2)
        cp3 = pltpu.async_copy(emb_hbm.at[pl.ds(base_e, CH)], emb_v, sem3)
        cp1.wait()
        cp2.wait()
        cp3.wait()

        def bbody(b, inner):
            col0 = b * 64
            for g in range(4):
                col = col0 + g * 16
                idx = src_v[0, pl.ds(col, 16)]
                sp = plsc.load_gather(sproj_v, [idx])
                s = sp + semb_v[pl.ds(col, 16)]
                e = jnp.exp(jnp.minimum(-s, (-ALPHA) * s))
                qoff = (idx & 7) * 16
                idx64_v[pl.ds(g * 16, 16)] = lax.shift_right_logical(idx, 3)
                qbuf_v[pl.ds(g * 16, 16)] = qoff
                for k in range(16):
                    i = col + k
                    ei = e[k]
                    for j in range(8):
                        emb_v[i, pl.ds(j * 16, 16)] = (
                            emb_v[i, pl.ds(j * 16, 16)] * ei)
                    evalue_v[g * 16 + k, pl.ds(qoff[k], 16)] = (
                        jnp.broadcast_to(ei, (16,)))
            pltpu.sync_copy(evalue_v, rs8_sh.at[idx64_v], add=True)
            for g in range(4):
                qv = qbuf_v[pl.ds(g * 16, 16)]
                for k in range(16):
                    evalue_v[g * 16 + k, pl.ds(qv[k], 16)] = zero16
            return inner

        lax.fori_loop(0, CH // 64, bbody, 0)

        pltpu.async_copy(emb_v, num_sh.at[src_v.at[0]], semn, add=True)
        return carry

    lax.fori_loop(0, nch, chunk_body, 0)
    pltpu.make_async_copy(emb_v, num_sh.at[src_v.at[0]], semn).wait()

    plsc.subcore_barrier()
    pltpu.sync_copy(num_sh.at[pl.ds(row0, ROWS_PER_TILE)],
                    num_out.at[cid, pl.ds(row0, ROWS_PER_TILE)])
    pltpu.sync_copy(rs8_sh.at[pl.ds(rrow0, RS_ROWS_PER_TILE)],
                    rs_out.at[cid, pl.ds(rrow0, RS_ROWS_PER_TILE)])

    @pl.when(sid == NTILE - 1)
    def _tail():
        r = NTILE * ROWS_PER_TILE
        pltpu.sync_copy(num_sh.at[pl.ds(r, N - r)],
                        num_out.at[cid, pl.ds(r, N - r)])


def _sc_segment(src2d, semb, sproj, emb_proj):
    mesh = plsc.VectorSubcoreMesh(core_axis_name="c", subcore_axis_name="s")
    f = pl.kernel(
        _sc_body,
        out_type=[
            jax.ShapeDtypeStruct((2, N, DOUT), jnp.float32),
            jax.ShapeDtypeStruct((2, NRS, 128), jnp.float32),
        ],
        mesh=mesh,
        compiler_params=pltpu.CompilerParams(needs_layout_passes=False),
        scratch_types=[
            pltpu.VMEM((N,), jnp.float32),
            pltpu.VMEM((NSUB, 128), jnp.int32),
            pltpu.VMEM((CH,), jnp.float32),
            pltpu.VMEM((CH, DOUT), jnp.float32),
            pltpu.VMEM((64, 128), jnp.float32),
            pltpu.VMEM((64,), jnp.int32),
            pltpu.VMEM((64,), jnp.int32),
            pltpu.SemaphoreType.DMA,
            pltpu.SemaphoreType.DMA,
            pltpu.SemaphoreType.DMA,
            pltpu.SemaphoreType.DMA,
            pltpu.VMEM_SHARED((N, DOUT), jnp.float32),
            pltpu.VMEM_SHARED((NRS, 128), jnp.float32),
        ],
    )
    return f(src2d, semb, sproj, emb_proj)


def _comb_body(h_ref, n0_ref, n1_ref, r0_ref, r1_ref, o_ref):
    rs = r0_ref[:, 0:1] + r1_ref[:, 0:1]
    num = n0_ref[...] + n1_ref[...]
    o_ref[...] = jnp.where(rs > 0.0, h_ref[...] + num / rs, 0.0)


def _combine(h_proj, num, rs, block_rows):
    grid = N // block_rows
    rspec = pl.BlockSpec((block_rows, DOUT), lambda i: (i, 0))
    sspec = pl.BlockSpec((block_rows, 16), lambda i: (i, 0))
    return pl.pallas_call(
        _comb_body,
        grid=(grid,),
        in_specs=[rspec, rspec, rspec, sspec, sspec],
        out_specs=rspec,
        out_shape=jax.ShapeDtypeStruct((N, DOUT), jnp.float32),
    )(h_proj, num[0], num[1], rs[0], rs[1])


@jax.jit
def kernel(input, edge, edge_embed, a, a_2):
    w1t = a[:, :DIN].T
    w2t = a[:, DIN:].T
    src3d = edge[0].reshape(NCHUNKS, NSUB, 128)

    h_proj, sproj = _projection(input, w1t, a_2, 2000)
    emb_proj, semb = _projection(edge_embed, w2t, a_2, 2560)

    num, rs8 = _sc_segment(src3d, semb.reshape(E), sproj.reshape(N), emb_proj)
    rs = rs8.reshape(2, NRS * 8, 16)[:, :N, :]
    return _combine(h_proj, num, rs, 2000)
